# baseline (speedup 1.0000x reference)
import functools

import jax
import jax.numpy as jnp
from jax import lax
from jax.experimental import pallas as pl
from jax.experimental.pallas import tpu as pltpu
from jax.experimental.pallas import tpu_sc as plsc

_N = 10000
_D = 128
_NC, _NS = 2, 16
_NW = _NC * _NS
_CHUNK = 128
_CPT = 80
_NCHUNKS = _NW * _CPT
_EPAD = _NCHUNKS * _CHUNK
_CPT0 = _NCHUNKS // _NS
_ROWS_PT = 640
_ACC_ROWS = _NS * _ROWS_PT
_TRASH = _N

_f32 = jnp.float32

_mesh = plsc.VectorSubcoreMesh(core_axis_name="c", subcore_axis_name="s")


def _sc_degree(dst3):
    @functools.partial(
        pl.kernel,
        out_type=jax.ShapeDtypeStruct((_NC, _ACC_ROWS, 16), _f32),
        mesh=_mesh,
        scratch_types=[
            pltpu.VMEM((_CPT, _CHUNK), jnp.int32),
            pltpu.VMEM((_CHUNK, 16), _f32),
            pltpu.VMEM((_CHUNK, 16), _f32),
            pltpu.VMEM_SHARED((_ACC_ROWS, 16), _f32),
        ],
    )
    def k(dst_hbm, out_hbm, idx_d, ones, zbuf, acc):
        c = lax.axis_index("c")
        s = lax.axis_index("s")
        w = c * _NS + s
        pltpu.sync_copy(dst_hbm.at[pl.ds(w * _CPT, _CPT)], idx_d)

        @pl.loop(0, _CHUNK)
        def _(i):
            ones[i, :] = jnp.full((16,), 1.0, _f32)
            zbuf[i, :] = jnp.zeros((16,), _f32)

        row0 = s * _ROWS_PT

        @pl.loop(0, _ROWS_PT // _CHUNK)
        def _(t):
            pltpu.sync_copy(zbuf, acc.at[pl.ds(row0 + t * _CHUNK, _CHUNK)])

        plsc.subcore_barrier()

        @pl.loop(0, _CPT)
        def _(j):
            pltpu.sync_copy(ones, acc.at[idx_d.at[j]], add=True)

        plsc.subcore_barrier()
        pltpu.sync_copy(acc.at[pl.ds(row0, _ROWS_PT)],
                        out_hbm.at[c, pl.ds(row0, _ROWS_PT)])

    return k(dst3)


_NBUF = 2
_SEG = 16
_ZROWS = 16


def _sc_aggregate(hs, src3, dst3):
    @functools.partial(
        pl.kernel,
        out_type=jax.ShapeDtypeStruct((_ACC_ROWS, _D), _f32),
        mesh=_mesh,
        scratch_types=[
            pltpu.VMEM((_SEG, _CHUNK), jnp.int32),
            pltpu.VMEM((_SEG, _CHUNK), jnp.int32),
        ] + [pltpu.VMEM((_CHUNK, _D), _f32)] * _NBUF + [
            pltpu.VMEM((_ZROWS, _D), _f32),
            pltpu.VMEM_SHARED((_ACC_ROWS, _D), _f32),
        ] + [pltpu.SemaphoreType.DMA] * _NBUF,
    )
    def k(hs_hbm, src_hbm, dst_hbm, out_hbm,
          idx_s, idx_d, *rest):
        rows = rest[:_NBUF]
        zbuf = rest[_NBUF]
        acc = rest[_NBUF + 1]
        sems = rest[_NBUF + 2:]
        c = lax.axis_index("c")
        s = lax.axis_index("s")
        row0 = s * _ROWS_PT
        base = s * _CPT0
        nseg = (1 - c) * (_CPT0 // _SEG)

        @pl.loop(0, _ZROWS)
        def _(i):
            @pl.loop(0, _D, step=16)
            def _(kk):
                zbuf[i, pl.ds(kk, 16)] = jnp.zeros((16,), _f32)

        @pl.loop(0, _ROWS_PT // _ZROWS)
        def _(t):
            pltpu.sync_copy(zbuf, acc.at[pl.ds(row0 + t * _ZROWS, _ZROWS)])

        plsc.subcore_barrier()

        @pl.loop(0, nseg)
        def _(seg):
            pltpu.sync_copy(
                src_hbm.at[pl.ds(base + seg * _SEG, _SEG)], idx_s)
            pltpu.sync_copy(
                dst_hbm.at[pl.ds(base + seg * _SEG, _SEG)], idx_d)

            for b in range(_NBUF):
                pltpu.async_copy(hs_hbm.at[idx_s.at[b]], rows[b], sems[b])

            @pl.loop(0, _SEG // _NBUF)
            def _(g):
                j0 = g * _NBUF
                for b in range(_NBUF):
                    j = j0 + b
                    pltpu.make_async_copy(
                        hs_hbm.at[idx_s.at[j]], rows[b], sems[b]).wait()
                    pltpu.sync_copy(rows[b], acc.at[idx_d.at[j]], add=True)

                    @pl.when(j + _NBUF < _SEG)
                    def _():
                        pltpu.async_copy(
                            hs_hbm.at[idx_s.at[j + _NBUF]], rows[b], sems[b])

        plsc.subcore_barrier()

        @pl.when(c == 0)
        def _():
            pltpu.sync_copy(acc.at[pl.ds(row0, _ROWS_PT)],
                            out_hbm.at[pl.ds(row0, _ROWS_PT)])

    return k(hs, src3, dst3)


def _tc_encoder(x, W, b, g, be):
    def body(x_ref, w_ref, b_ref, g_ref, be_ref, o_ref):
        h = jnp.dot(x_ref[...], w_ref[...], preferred_element_type=_f32)
        h = h + b_ref[...]
        m = jnp.mean(h, axis=0, keepdims=True)
        v = jnp.mean((h - m) * (h - m), axis=0, keepdims=True)
        h = (h - m) * lax.rsqrt(v + 1e-5) * g_ref[...] + be_ref[...]
        o_ref[...] = jnp.maximum(h, 0.0)

    return pl.pallas_call(
        body, out_shape=jax.ShapeDtypeStruct((_N, _D), _f32))(
            x, W, b.reshape(1, _D), g.reshape(1, _D), be.reshape(1, _D))


def _tc_dinv(degp):
    def body(p_ref, o_ref):
        d = 1.0 + p_ref[0, :_N, :1] + p_ref[1, :_N, :1]
        o_ref[...] = lax.rsqrt(d)

    return pl.pallas_call(
        body, out_shape=jax.ShapeDtypeStruct((_N, 1), _f32))(degp)


def _store_padded(o_ref, hs):
    o_ref[: _N, :] = hs
    o_ref[_N:, :] = jnp.zeros((_ACC_ROWS - _N, _D), _f32)


_HS_T = jax.ShapeDtypeStruct((_ACC_ROWS, _D), _f32)


def _tc_mm_scale(h, W, dinv):
    def body(h_ref, w_ref, di_ref, o_ref):
        _store_padded(o_ref, jnp.dot(h_ref[...], w_ref[...],
                                     preferred_element_type=_f32) * di_ref[...])

    return pl.pallas_call(body, out_shape=_HS_T)(h, W, dinv)


def _tc_comb_mm(p, hs, bvec, dinv, W):
    def body(p_ref, hs_ref, b_ref, di_ref, w_ref, o_ref):
        h = (p_ref[:_N, :] + hs_ref[:_N, :]) * di_ref[...]
        h = h + b_ref[...]
        _store_padded(o_ref, jnp.dot(h, w_ref[...],
                                     preferred_element_type=_f32) * di_ref[...])

    return pl.pallas_call(body, out_shape=_HS_T)(
        p, hs, bvec.reshape(1, _D), dinv, W)


def _tc_epilogue(p, hs, bvec, dinv):
    def body(p_ref, hs_ref, b_ref, di_ref, o_ref):
        h = (p_ref[:_N, :] + hs_ref[:_N, :]) * di_ref[...]
        o_ref[...] = h + b_ref[...]

    return pl.pallas_call(
        body, out_shape=jax.ShapeDtypeStruct((_N, _D), _f32))(
            p, hs, bvec.reshape(1, _D), dinv)


def kernel(x, adj_t, W_enc, b_enc, gamma, beta, Wc0, bc0, Wc1, bc1, Wc2, bc2):
    src = adj_t[0]
    dst = adj_t[1]
    pad = _EPAD - src.shape[0]
    src3 = jnp.concatenate(
        [src, jnp.zeros((pad,), jnp.int32)]).reshape(_NCHUNKS, _CHUNK)
    trash = _TRASH + (jnp.arange(pad, dtype=jnp.int32) % (_ACC_ROWS - _N))
    dst3 = jnp.concatenate([dst, trash]).reshape(_NCHUNKS, _CHUNK)

    degp = _sc_degree(dst3)
    h0 = _tc_encoder(x, W_enc, b_enc, gamma, beta)
    dinv = _tc_dinv(degp)

    hs0 = _tc_mm_scale(h0, Wc0, dinv)
    p0 = _sc_aggregate(hs0, src3, dst3)
    hs1 = _tc_comb_mm(p0, hs0, bc0, dinv, Wc1)
    p1 = _sc_aggregate(hs1, src3, dst3)
    hs2 = _tc_comb_mm(p1, hs1, bc1, dinv, Wc2)
    p2 = _sc_aggregate(hs2, src3, dst3)
    return _tc_epilogue(p2, hs2, bc2, dinv)


# speedup vs baseline: 1.4195x; 1.4195x over previous
import functools

import jax
import jax.numpy as jnp
from jax import lax
from jax.experimental import pallas as pl
from jax.experimental.pallas import tpu as pltpu
from jax.experimental.pallas import tpu_sc as plsc

_N = 10000
_D = 128
_NC, _NS = 2, 16
_NW = _NC * _NS
_CHUNK = 128
_CPT = 80
_NCHUNKS = _NW * _CPT
_EPAD = _NCHUNKS * _CHUNK
_CPT0, _CPT1 = 144, 16
_ROWS_PT = 640
_ACC_ROWS = _NS * _ROWS_PT
_TRASH = _N

_f32 = jnp.float32

_mesh = plsc.VectorSubcoreMesh(core_axis_name="c", subcore_axis_name="s")


def _sc_degree(dst3):
    @functools.partial(
        pl.kernel,
        out_type=jax.ShapeDtypeStruct((_NC, _ACC_ROWS, 16), _f32),
        mesh=_mesh,
        scratch_types=[
            pltpu.VMEM((_CPT, _CHUNK), jnp.int32),
            pltpu.VMEM((_CHUNK, 16), _f32),
            pltpu.VMEM((_CHUNK, 16), _f32),
            pltpu.VMEM_SHARED((_ACC_ROWS, 16), _f32),
        ],
    )
    def k(dst_hbm, out_hbm, idx_d, ones, zbuf, acc):
        c = lax.axis_index("c")
        s = lax.axis_index("s")
        w = c * _NS + s
        pltpu.sync_copy(dst_hbm.at[pl.ds(w * _CPT, _CPT)], idx_d)

        @pl.loop(0, _CHUNK)
        def _(i):
            ones[i, :] = jnp.full((16,), 1.0, _f32)
            zbuf[i, :] = jnp.zeros((16,), _f32)

        row0 = s * _ROWS_PT

        @pl.loop(0, _ROWS_PT // _CHUNK)
        def _(t):
            pltpu.sync_copy(zbuf, acc.at[pl.ds(row0 + t * _CHUNK, _CHUNK)])

        plsc.subcore_barrier()

        @pl.loop(0, _CPT)
        def _(j):
            pltpu.sync_copy(ones, acc.at[idx_d.at[j]], add=True)

        plsc.subcore_barrier()
        pltpu.sync_copy(acc.at[pl.ds(row0, _ROWS_PT)],
                        out_hbm.at[c, pl.ds(row0, _ROWS_PT)])

    return k(dst3)


_NBUF = 2
_SEG = 16
_ZROWS = 16


def _sc_aggregate(hs, src3, dst3):
    @functools.partial(
        pl.kernel,
        out_type=jax.ShapeDtypeStruct((_NC, _ACC_ROWS, _D), _f32),
        mesh=_mesh,
        scratch_types=[
            pltpu.VMEM((_SEG, _CHUNK), jnp.int32),
            pltpu.VMEM((_SEG, _CHUNK), jnp.int32),
        ] + [pltpu.VMEM((_CHUNK, _D), _f32)] * _NBUF + [
            pltpu.VMEM((_ZROWS, _D), _f32),
            pltpu.VMEM_SHARED((_ACC_ROWS, _D), _f32),
        ] + [pltpu.SemaphoreType.DMA] * _NBUF,
    )
    def k(hs_hbm, src_hbm, dst_hbm, out_hbm,
          idx_s, idx_d, *rest):
        rows = rest[:_NBUF]
        zbuf = rest[_NBUF]
        acc = rest[_NBUF + 1]
        sems = rest[_NBUF + 2:]
        c = lax.axis_index("c")
        s = lax.axis_index("s")
        row0 = s * _ROWS_PT
        base = (1 - c) * (s * _CPT0) + c * (_NS * _CPT0 + s * _CPT1)
        nseg = ((1 - c) * _CPT0 + c * _CPT1) // _SEG

        @pl.loop(0, _ZROWS)
        def _(i):
            @pl.loop(0, _D, step=16)
            def _(kk):
                zbuf[i, pl.ds(kk, 16)] = jnp.zeros((16,), _f32)

        @pl.loop(0, _ROWS_PT // _ZROWS)
        def _(t):
            pltpu.sync_copy(zbuf, acc.at[pl.ds(row0 + t * _ZROWS, _ZROWS)])

        plsc.subcore_barrier()

        @pl.loop(0, nseg)
        def _(seg):
            pltpu.sync_copy(
                src_hbm.at[pl.ds(base + seg * _SEG, _SEG)], idx_s)
            pltpu.sync_copy(
                dst_hbm.at[pl.ds(base + seg * _SEG, _SEG)], idx_d)

            for b in range(_NBUF):
                pltpu.async_copy(hs_hbm.at[idx_s.at[b]], rows[b], sems[b])

            @pl.loop(0, _SEG // _NBUF)
            def _(g):
                j0 = g * _NBUF
                for b in range(_NBUF):
                    j = j0 + b
                    pltpu.make_async_copy(
                        hs_hbm.at[idx_s.at[j]], rows[b], sems[b]).wait()
                    pltpu.sync_copy(rows[b], acc.at[idx_d.at[j]], add=True)

                    @pl.when(j + _NBUF < _SEG)
                    def _():
                        pltpu.async_copy(
                            hs_hbm.at[idx_s.at[j + _NBUF]], rows[b], sems[b])

        plsc.subcore_barrier()
        pltpu.sync_copy(acc.at[pl.ds(row0, _ROWS_PT)],
                        out_hbm.at[c, pl.ds(row0, _ROWS_PT)])

    return k(hs, src3, dst3)


def _tc_encoder(x, W, b, g, be):
    def body(x_ref, w_ref, b_ref, g_ref, be_ref, o_ref):
        h = jnp.dot(x_ref[...], w_ref[...], preferred_element_type=_f32)
        h = h + b_ref[...]
        m = jnp.mean(h, axis=0, keepdims=True)
        v = jnp.mean((h - m) * (h - m), axis=0, keepdims=True)
        h = (h - m) * lax.rsqrt(v + 1e-5) * g_ref[...] + be_ref[...]
        o_ref[...] = jnp.maximum(h, 0.0)

    return pl.pallas_call(
        body, out_shape=jax.ShapeDtypeStruct((_N, _D), _f32))(
            x, W, b.reshape(1, _D), g.reshape(1, _D), be.reshape(1, _D))


def _tc_dinv(degp):
    def body(p_ref, o_ref):
        d = 1.0 + p_ref[0, :_N, :1] + p_ref[1, :_N, :1]
        o_ref[...] = lax.rsqrt(d)

    return pl.pallas_call(
        body, out_shape=jax.ShapeDtypeStruct((_N, 1), _f32))(degp)


def _store_padded(o_ref, hs):
    o_ref[: _N, :] = hs
    o_ref[_N:, :] = jnp.zeros((_ACC_ROWS - _N, _D), _f32)


_HS_T = jax.ShapeDtypeStruct((_ACC_ROWS, _D), _f32)


def _tc_mm_scale(h, W, dinv):
    def body(h_ref, w_ref, di_ref, o_ref):
        _store_padded(o_ref, jnp.dot(h_ref[...], w_ref[...],
                                     preferred_element_type=_f32) * di_ref[...])

    return pl.pallas_call(body, out_shape=_HS_T)(h, W, dinv)


def _tc_comb_mm(p, hs, bvec, dinv, W):
    def body(p_ref, hs_ref, b_ref, di_ref, w_ref, o_ref):
        h = (p_ref[0, :_N, :] + p_ref[1, :_N, :] + hs_ref[:_N, :]) * di_ref[...]
        h = h + b_ref[...]
        _store_padded(o_ref, jnp.dot(h, w_ref[...],
                                     preferred_element_type=_f32) * di_ref[...])

    return pl.pallas_call(body, out_shape=_HS_T)(
        p, hs, bvec.reshape(1, _D), dinv, W)


def _tc_epilogue(p, hs, bvec, dinv):
    def body(p_ref, hs_ref, b_ref, di_ref, o_ref):
        h = (p_ref[0, :_N, :] + p_ref[1, :_N, :] + hs_ref[:_N, :]) * di_ref[...]
        o_ref[...] = h + b_ref[...]

    return pl.pallas_call(
        body, out_shape=jax.ShapeDtypeStruct((_N, _D), _f32))(
            p, hs, bvec.reshape(1, _D), dinv)


def kernel(x, adj_t, W_enc, b_enc, gamma, beta, Wc0, bc0, Wc1, bc1, Wc2, bc2):
    src = adj_t[0]
    dst = adj_t[1]
    pad = _EPAD - src.shape[0]
    src3 = jnp.concatenate(
        [src, jnp.zeros((pad,), jnp.int32)]).reshape(_NCHUNKS, _CHUNK)
    trash = _TRASH + (jnp.arange(pad, dtype=jnp.int32) % (_ACC_ROWS - _N))
    dst3 = jnp.concatenate([dst, trash]).reshape(_NCHUNKS, _CHUNK)

    degp = _sc_degree(dst3)
    h0 = _tc_encoder(x, W_enc, b_enc, gamma, beta)
    dinv = _tc_dinv(degp)

    hs0 = _tc_mm_scale(h0, Wc0, dinv)
    p0 = _sc_aggregate(hs0, src3, dst3)
    hs1 = _tc_comb_mm(p0, hs0, bc0, dinv, Wc1)
    p1 = _sc_aggregate(hs1, src3, dst3)
    hs2 = _tc_comb_mm(p1, hs1, bc1, dinv, Wc2)
    p2 = _sc_aggregate(hs2, src3, dst3)
    return _tc_epilogue(p2, hs2, bc2, dinv)


# speedup vs baseline: 2.4849x; 1.7506x over previous
import functools

import jax
import jax.numpy as jnp
from jax import lax
from jax.experimental import pallas as pl
from jax.experimental.pallas import tpu as pltpu
from jax.experimental.pallas import tpu_sc as plsc

_N = 10000
_D = 128
_NC, _NS = 2, 16
_NW = _NC * _NS
_CHUNK = 128
_CPT = 80
_NCHUNKS = _NW * _CPT
_EPAD = _NCHUNKS * _CHUNK
_CPT0, _CPT1 = 144, 16
_ROWS_PT = 640
_ACC_ROWS = _NS * _ROWS_PT
_TRASH = _N

_f32 = jnp.float32

_mesh = plsc.VectorSubcoreMesh(core_axis_name="c", subcore_axis_name="s")


def _sc_degree(dst3):
    @functools.partial(
        pl.kernel,
        out_type=jax.ShapeDtypeStruct((_NC, _ACC_ROWS, 16), _f32),
        mesh=_mesh,
        scratch_types=[
            pltpu.VMEM((_CPT, _CHUNK), jnp.int32),
            pltpu.VMEM((_CHUNK, 16), _f32),
            pltpu.VMEM((_CHUNK, 16), _f32),
            pltpu.VMEM_SHARED((_ACC_ROWS, 16), _f32),
        ],
    )
    def k(dst_hbm, out_hbm, idx_d, ones, zbuf, acc):
        c = lax.axis_index("c")
        s = lax.axis_index("s")
        w = c * _NS + s
        pltpu.sync_copy(dst_hbm.at[pl.ds(w * _CPT, _CPT)], idx_d)

        @pl.loop(0, _CHUNK)
        def _(i):
            ones[i, :] = jnp.full((16,), 1.0, _f32)
            zbuf[i, :] = jnp.zeros((16,), _f32)

        row0 = s * _ROWS_PT

        @pl.loop(0, _ROWS_PT // _CHUNK)
        def _(t):
            pltpu.sync_copy(zbuf, acc.at[pl.ds(row0 + t * _CHUNK, _CHUNK)])

        plsc.subcore_barrier()

        @pl.loop(0, _CPT)
        def _(j):
            pltpu.sync_copy(ones, acc.at[idx_d.at[j]], add=True)

        plsc.subcore_barrier()
        pltpu.sync_copy(acc.at[pl.ds(row0, _ROWS_PT)],
                        out_hbm.at[c, pl.ds(row0, _ROWS_PT)])

    return k(dst3)


_NBUF = 2
_SEG = 16
_ZROWS = 16


def _sc_aggregate(hs, src3, dst3):
    @functools.partial(
        pl.kernel,
        out_type=jax.ShapeDtypeStruct((_NC, _ACC_ROWS, _D), _f32),
        mesh=_mesh,
        scratch_types=[
            pltpu.VMEM((_SEG, _CHUNK), jnp.int32),
            pltpu.VMEM((_SEG, _CHUNK), jnp.int32),
        ] + [pltpu.VMEM((_CHUNK, _D), _f32)] * _NBUF + [
            pltpu.VMEM((_ZROWS, _D), _f32),
            pltpu.VMEM_SHARED((_ACC_ROWS, _D), _f32),
        ] + [pltpu.SemaphoreType.DMA] * _NBUF,
    )
    def k(hs_hbm, src_hbm, dst_hbm, out_hbm,
          idx_s, idx_d, *rest):
        rows = rest[:_NBUF]
        zbuf = rest[_NBUF]
        acc = rest[_NBUF + 1]
        sems = rest[_NBUF + 2:]
        c = lax.axis_index("c")
        s = lax.axis_index("s")
        row0 = s * _ROWS_PT
        base = (1 - c) * (s * _CPT0) + c * (_NS * _CPT0 + s * _CPT1)
        nseg = ((1 - c) * _CPT0 + c * _CPT1) // _SEG

        @pl.loop(0, _ZROWS)
        def _(i):
            @pl.loop(0, _D, step=16)
            def _(kk):
                zbuf[i, pl.ds(kk, 16)] = jnp.zeros((16,), _f32)

        @pl.loop(0, _ROWS_PT // _ZROWS)
        def _(t):
            pltpu.sync_copy(zbuf, acc.at[pl.ds(row0 + t * _ZROWS, _ZROWS)])

        plsc.subcore_barrier()

        @pl.loop(0, nseg)
        def _(seg):
            pltpu.sync_copy(
                src_hbm.at[pl.ds(base + seg * _SEG, _SEG)], idx_s)
            pltpu.sync_copy(
                dst_hbm.at[pl.ds(base + seg * _SEG, _SEG)], idx_d)

            for b in range(_NBUF):
                pltpu.async_copy(hs_hbm.at[idx_s.at[b]], rows[b], sems[b])

            @pl.loop(0, _SEG // _NBUF)
            def _(g):
                j0 = g * _NBUF
                for b in range(_NBUF):
                    j = j0 + b
                    pltpu.make_async_copy(
                        hs_hbm.at[idx_s.at[j]], rows[b], sems[b]).wait()
                    pltpu.sync_copy(rows[b], acc.at[idx_d.at[j]], add=True)

                    @pl.when(j + _NBUF < _SEG)
                    def _():
                        pltpu.async_copy(
                            hs_hbm.at[idx_s.at[j + _NBUF]], rows[b], sems[b])

        plsc.subcore_barrier()
        pltpu.sync_copy(acc.at[pl.ds(row0, _ROWS_PT)],
                        out_hbm.at[c, pl.ds(row0, _ROWS_PT)])

    return k(hs, src3, dst3)


def _tc_encoder(x, W, b, g, be):
    def body(x_ref, w_ref, b_ref, g_ref, be_ref, o_ref):
        h = jnp.dot(x_ref[...], w_ref[...], preferred_element_type=_f32)
        h = h + b_ref[...]
        m = jnp.mean(h, axis=0, keepdims=True)
        v = jnp.mean((h - m) * (h - m), axis=0, keepdims=True)
        h = (h - m) * lax.rsqrt(v + 1e-5) * g_ref[...] + be_ref[...]
        o_ref[...] = jnp.maximum(h, 0.0)

    return pl.pallas_call(
        body, out_shape=jax.ShapeDtypeStruct((_N, _D), _f32))(
            x, W, b.reshape(1, _D), g.reshape(1, _D), be.reshape(1, _D))


def _tc_dinv(degp):
    def body(p_ref, o_ref):
        d = 1.0 + p_ref[0, :_N, :1] + p_ref[1, :_N, :1]
        o_ref[...] = lax.rsqrt(d)

    return pl.pallas_call(
        body, out_shape=jax.ShapeDtypeStruct((_N, 1), _f32))(degp)


def _store_padded(o_ref, hs):
    o_ref[: _N, :] = hs
    o_ref[_N:, :] = jnp.zeros((_ACC_ROWS - _N, _D), _f32)


_HS_T = jax.ShapeDtypeStruct((_ACC_ROWS, _D), _f32)


def _tc_mm_scale(h, W, dinv):
    def body(h_ref, w_ref, di_ref, o_ref):
        _store_padded(o_ref, jnp.dot(h_ref[...], w_ref[...],
                                     preferred_element_type=_f32) * di_ref[...])

    return pl.pallas_call(body, out_shape=_HS_T)(h, W, dinv)


def _tc_comb_mm(p, hs, bvec, dinv, W):
    def body(p_ref, hs_ref, b_ref, di_ref, w_ref, o_ref):
        h = (p_ref[0, :_N, :] + p_ref[1, :_N, :] + hs_ref[:_N, :]) * di_ref[...]
        h = h + b_ref[...]
        _store_padded(o_ref, jnp.dot(h, w_ref[...],
                                     preferred_element_type=_f32) * di_ref[...])

    return pl.pallas_call(body, out_shape=_HS_T)(
        p, hs, bvec.reshape(1, _D), dinv, W)


def _tc_epilogue(p, hs, bvec, dinv):
    def body(p_ref, hs_ref, b_ref, di_ref, o_ref):
        h = (p_ref[0, :_N, :] + p_ref[1, :_N, :] + hs_ref[:_N, :]) * di_ref[...]
        o_ref[...] = h + b_ref[...]

    return pl.pallas_call(
        body, out_shape=jax.ShapeDtypeStruct((_N, _D), _f32))(
            p, hs, bvec.reshape(1, _D), dinv)


def kernel(x, adj_t, W_enc, b_enc, gamma, beta, Wc0, bc0, Wc1, bc1, Wc2, bc2):
    src = adj_t[0]
    dst = adj_t[1]
    pad = _EPAD - src.shape[0]
    src_pad = jnp.arange(pad, dtype=jnp.int32) % _N
    src3 = jnp.concatenate([src, src_pad]).reshape(_NCHUNKS, _CHUNK)
    trash = _TRASH + (jnp.arange(pad, dtype=jnp.int32) % (_ACC_ROWS - _N))
    dst3 = jnp.concatenate([dst, trash]).reshape(_NCHUNKS, _CHUNK)

    degp = _sc_degree(dst3)
    h0 = _tc_encoder(x, W_enc, b_enc, gamma, beta)
    dinv = _tc_dinv(degp)

    hs0 = _tc_mm_scale(h0, Wc0, dinv)
    p0 = _sc_aggregate(hs0, src3, dst3)
    hs1 = _tc_comb_mm(p0, hs0, bc0, dinv, Wc1)
    p1 = _sc_aggregate(hs1, src3, dst3)
    hs2 = _tc_comb_mm(p1, hs1, bc1, dinv, Wc2)
    p2 = _sc_aggregate(hs2, src3, dst3)
    return _tc_epilogue(p2, hs2, bc2, dinv)
